# baseline (speedup 1.0000x reference)
import jax
import jax.numpy as jnp
from jax.experimental import pallas as pl

N = 10000
D = 128
H1 = 64
H2 = 32

_BLK = 1000


def _fused_kernel(x_ref,
                  wz1_ref, bz1_ref, wh1_ref, bh1_ref,
                  wz2_ref, bz2_ref, wh2_ref, bh2_ref,
                  wl1_ref, bl1_ref, wl2_ref, bl2_ref,
                  out_ref):
    bf16 = jnp.bfloat16
    x = x_ref[...].astype(bf16)

    a1 = (wz1_ref[0, :D, :] + wz1_ref[1, :D, :]) * 0.5
    b1 = wh1_ref[0, :D, :] + wh1_ref[1, :D, :]
    w1 = jnp.concatenate([a1, b1], axis=1).astype(bf16)
    bias1 = jnp.concatenate([bz1_ref[...] * 0.5, bh1_ref[...]], axis=1)
    u1 = jnp.dot(x, w1, preferred_element_type=jnp.float32) + bias1
    t1 = jnp.tanh(u1)
    g1 = jax.nn.relu((1.0 - t1[:, :H1]) * t1[:, H1:])

    a2 = (wz2_ref[0, :H1, :] + wz2_ref[1, :H1, :]) * 0.25
    b2 = (wh2_ref[0, :H1, :] + wh2_ref[1, :H1, :]) * 0.5
    w2 = jnp.concatenate([a2, b2], axis=1).astype(bf16)
    bias2 = jnp.concatenate([bz2_ref[...] * 0.5, bh2_ref[...]], axis=1)
    u2 = jnp.dot(g1.astype(bf16), w2, preferred_element_type=jnp.float32) + bias2
    t2 = jnp.tanh(u2)
    g2 = jax.nn.relu((1.0 - t2[:, :H2]) * t2[:, H2:])

    h3 = jax.nn.relu(
        jnp.dot(g2, wl1_ref[...] * 0.5, preferred_element_type=jnp.float32)
        + bl1_ref[...])
    out_ref[...] = (jnp.dot(h3, wl2_ref[...],
                            preferred_element_type=jnp.float32)
                    + bl2_ref[...])


def kernel(x, edge_index, edge_weight,
           W_z1, b_z1, W_r1, b_r1, W_h1, b_h1,
           W_z2, b_z2, W_r2, b_r2, W_h2, b_h2,
           W_l1, b_l1, W_l2, b_l2):
    del edge_index, edge_weight, W_r1, b_r1, W_r2, b_r2

    def wspec(a):
        shp = a.shape
        return pl.BlockSpec(shp, lambda i: (0,) * len(shp))

    biases = [b.reshape(1, -1) for b in (b_z1, b_h1, b_z2, b_h2, b_l1, b_l2)]
    bz1, bh1, bz2, bh2, bl1, bl2 = biases

    out = pl.pallas_call(
        _fused_kernel,
        grid=(N // _BLK,),
        in_specs=[
            pl.BlockSpec((_BLK, D), lambda i: (i, 0)),
            wspec(W_z1), wspec(bz1), wspec(W_h1), wspec(bh1),
            wspec(W_z2), wspec(bz2), wspec(W_h2), wspec(bh2),
            wspec(W_l1), wspec(bl1), wspec(W_l2), wspec(bl2),
        ],
        out_specs=pl.BlockSpec((_BLK, 1), lambda i: (i, 0)),
        out_shape=jax.ShapeDtypeStruct((N, 1), jnp.float32),
    )(x, W_z1, bz1, W_h1, bh1, W_z2, bz2, W_h2, bh2, W_l1, bl1, W_l2, bl2)
    return out


# speedup vs baseline: 1.1754x; 1.1754x over previous
import jax
import jax.numpy as jnp
from jax.experimental import pallas as pl

N = 10000
D = 128
H1 = 64
H2 = 32

_BLK = 2000


def _fused_kernel(x_ref,
                  wz1_ref, bz1_ref, wh1_ref, bh1_ref,
                  wz2_ref, bz2_ref, wh2_ref, bh2_ref,
                  wl1_ref, bl1_ref, wl2_ref, bl2_ref,
                  out_ref):
    bf16 = jnp.bfloat16
    x = x_ref[...].astype(bf16)

    a1 = (wz1_ref[0, :D, :] + wz1_ref[1, :D, :]) * 0.5
    b1 = wh1_ref[0, :D, :] + wh1_ref[1, :D, :]
    w1 = jnp.concatenate([a1, b1], axis=1).astype(bf16)
    bias1 = jnp.concatenate([bz1_ref[...] * 0.5, bh1_ref[...]], axis=1)
    u1 = jnp.dot(x, w1, preferred_element_type=jnp.float32) + bias1
    t1 = jnp.tanh(u1)
    g1 = jax.nn.relu((1.0 - t1[:, :H1]) * t1[:, H1:])

    a2 = (wz2_ref[0, :H1, :] + wz2_ref[1, :H1, :]) * 0.25
    b2 = (wh2_ref[0, :H1, :] + wh2_ref[1, :H1, :]) * 0.5
    w2 = jnp.concatenate([a2, b2], axis=1).astype(bf16)
    bias2 = jnp.concatenate([bz2_ref[...] * 0.5, bh2_ref[...]], axis=1)
    u2 = jnp.dot(g1.astype(bf16), w2, preferred_element_type=jnp.float32) + bias2
    t2 = jnp.tanh(u2)
    g2 = jax.nn.relu((1.0 - t2[:, :H2]) * t2[:, H2:])

    h3 = jax.nn.relu(
        jnp.dot(g2, wl1_ref[...] * 0.5, preferred_element_type=jnp.float32)
        + bl1_ref[...])
    out_ref[...] = (jnp.dot(h3, wl2_ref[...],
                            preferred_element_type=jnp.float32)
                    + bl2_ref[...])


def kernel(x, edge_index, edge_weight,
           W_z1, b_z1, W_r1, b_r1, W_h1, b_h1,
           W_z2, b_z2, W_r2, b_r2, W_h2, b_h2,
           W_l1, b_l1, W_l2, b_l2):
    del edge_index, edge_weight, W_r1, b_r1, W_r2, b_r2

    def wspec(a):
        shp = a.shape
        return pl.BlockSpec(shp, lambda i: (0,) * len(shp))

    biases = [b.reshape(1, -1) for b in (b_z1, b_h1, b_z2, b_h2, b_l1, b_l2)]
    bz1, bh1, bz2, bh2, bl1, bl2 = biases

    out = pl.pallas_call(
        _fused_kernel,
        grid=(N // _BLK,),
        in_specs=[
            pl.BlockSpec((_BLK, D), lambda i: (i, 0)),
            wspec(W_z1), wspec(bz1), wspec(W_h1), wspec(bh1),
            wspec(W_z2), wspec(bz2), wspec(W_h2), wspec(bh2),
            wspec(W_l1), wspec(bl1), wspec(W_l2), wspec(bl2),
        ],
        out_specs=pl.BlockSpec((_BLK, 1), lambda i: (i, 0)),
        out_shape=jax.ShapeDtypeStruct((N, 1), jnp.float32),
    )(x, W_z1, bz1, W_h1, bh1, W_z2, bz2, W_h2, bh2, W_l1, bl1, W_l2, bl2)
    return out
